# baseline (speedup 1.0000x reference)
import numpy as np

import jax
import jax.numpy as jnp
from jax import lax
from jax.experimental import pallas as pl
from jax.experimental.pallas import tpu as pltpu
from jax.experimental.pallas import tpu_sc as plsc

_G = 512
_N = 100000
_NW = 32
_CHUNK = 3136
_NVEC = _CHUNK // 16
_LAST_BASE = _N - _CHUNK
_LAST_SKIP = (31 * _CHUNK - _LAST_BASE) // 16
_NSEG = 544
_PITCH = 545

_DIMS = (119, 4, 12, 12, 10, 6, 6, 2, 2)
_OFFS = (0, 119, 123, 135, 147, 157, 163, 169, 171)

_C012 = 119 * 48
_C345 = 12 * 64
_C678 = 32


def _project_body(t0, t1, t2, t3, t4, t5, t6, t7, t8, w, p_ref):
    ts = (t0, t1, t2, t3, t4, t5, t6, t7, t8)
    for t, off, d in zip(ts, _OFFS, _DIMS):
        p_ref[pl.ds(off, d)] = jnp.dot(t[...], w[...])[:, 0]


_CONST_ROWS = []


def _row(vals):
    _CONST_ROWS.append(np.asarray(vals, np.int32))
    return len(_CONST_ROWS) - 1


_R12 = []
for _v in range(3):
    _kk = np.arange(16) + 16 * _v
    _R12.append((_row(_kk // 12 + _OFFS[1]), _row(_kk % 12 + _OFFS[2])))
_R45 = []
for _v in range(4):
    _kk = np.arange(16) + 16 * _v
    _R45.append((_row(np.minimum(_kk // 6, 9) + _OFFS[4]),
                 _row(_kk % 6 + _OFFS[5])))
_R678 = []
for _v in range(2):
    _kk = np.arange(16) + 16 * _v
    _R678.append((_row(np.minimum(_kk // 4, 5) + _OFFS[6]),
                  _row((_kk // 2) % 2 + _OFFS[7]),
                  _row(_kk % 2 + _OFFS[8])))
_CONSTS_NP = np.concatenate(_CONST_ROWS)


def _pool_body(x_hbm, b_hbm, p_hbm, c_hbm, out,
               xv, bv, pv, cv, p12, p45, t012, t345, t678,
               acc_s, acc_c, obuf, sem_big, sem_small):
    wid = lax.axis_index("c") * 16 + lax.axis_index("s")
    is_last = wid == _NW - 1
    base = jnp.where(is_last, _LAST_BASE, wid * _CHUNK)

    cp_x = pltpu.async_copy(x_hbm.at[pl.ds(base * 9, _CHUNK * 9)], xv,
                            sem_big)
    cp_b = pltpu.async_copy(b_hbm.at[pl.ds(base, _CHUNK)], bv, sem_big)
    cp_p = pltpu.async_copy(p_hbm, pv, sem_small)
    cp_c = pltpu.async_copy(c_hbm, cv, sem_small)
    cp_p.wait()
    cp_c.wait()

    iota = lax.iota(jnp.int32, 16)
    zeros = jnp.zeros((16,), jnp.float32)

    def crow(r):
        return cv[pl.ds(16 * r, 16)]

    for v, (ra, rc) in enumerate(_R12):
        p12[pl.ds(16 * v, 16)] = (plsc.load_gather(pv, [crow(ra)])
                                  + plsc.load_gather(pv, [crow(rc)]))
    for v, (ra, rc) in enumerate(_R45):
        p45[pl.ds(16 * v, 16)] = (plsc.load_gather(pv, [crow(ra)])
                                  + plsc.load_gather(pv, [crow(rc)]))
    for v, (ra, rb, rc) in enumerate(_R678):
        t678[pl.ds(16 * v, 16)] = (plsc.load_gather(pv, [crow(ra)])
                                   + plsc.load_gather(pv, [crow(rb)])
                                   + plsc.load_gather(pv, [crow(rc)]))

    def c012_body(a, _):
        s = plsc.load_gather(pv, [jnp.zeros((16,), jnp.int32) + a])
        r = a * 48
        t012[pl.ds(r, 16)] = s + p12[pl.ds(0, 16)]
        t012[pl.ds(r + 16, 16)] = s + p12[pl.ds(16, 16)]
        t012[pl.ds(r + 32, 16)] = s + p12[pl.ds(32, 16)]
        return 0

    lax.fori_loop(0, 119, c012_body, 0)

    def c345_body(a, _):
        s = plsc.load_gather(pv, [jnp.zeros((16,), jnp.int32) + (a + _OFFS[3])])
        r = a * 64
        for v in range(4):
            t345[pl.ds(r + 16 * v, 16)] = s + p45[pl.ds(16 * v, 16)]
        return 0

    lax.fori_loop(0, 12, c345_body, 0)

    def zero_body(k, _):
        acc_s[pl.ds(k * 16, 16)] = zeros
        acc_c[pl.ds(k * 16, 16)] = zeros
        return 0

    lax.fori_loop(0, _PITCH, zero_body, 0)

    cp_x.wait()
    cp_b.wait()

    ones = jnp.full((16,), 1.0, jnp.float32)
    lane_base = iota * _PITCH
    nine = iota * 9

    def step(j):
        b0 = nine + j * 144
        xf = [plsc.load_gather(xv, [b0 + f]) for f in range(9)]
        i012 = xf[0] * 48 + xf[1] * 12 + xf[2]
        i345 = xf[3] * 64 + xf[4] * 6 + xf[5]
        i678 = xf[6] * 4 + xf[7] * 2 + xf[8]
        s = plsc.load_gather(t012, [i012])
        s = s + plsc.load_gather(t345, [i345])
        s = s + plsc.load_gather(t678, [i678])
        g = bv[pl.ds(j * 16, 16)] + lane_base
        plsc.addupdate_scatter(acc_s, [g], s)
        plsc.addupdate_scatter(acc_c, [g], ones)

    def body(t, _):
        step(2 * t)
        step(2 * t + 1)
        return 0

    lax.cond(is_last,
             lambda: lax.fori_loop(_LAST_SKIP // 2, _NVEC // 2, body, 0),
             lambda: lax.fori_loop(0, _NVEC // 2, body, 0))

    def red_body(k, _):
        off = k * 16
        ssum = acc_s[pl.ds(off, 16)]
        csum = acc_c[pl.ds(off, 16)]
        for l in range(1, 16):
            ssum = ssum + acc_s[pl.ds(l * _PITCH + off, 16)]
            csum = csum + acc_c[pl.ds(l * _PITCH + off, 16)]
        obuf[pl.ds(off, 16)] = ssum
        obuf[pl.ds(_NSEG + off, 16)] = csum
        return 0

    lax.fori_loop(0, _NSEG // 16, red_body, 0)

    pltpu.sync_copy(obuf, out.at[wid])


def _finish_body(p_ref, b_ref, o_ref):
    tot = jnp.sum(p_ref[...], axis=0)
    sums = tot[:_G]
    counts = tot[_NSEG:_NSEG + _G]
    o_ref[...] = (sums / jnp.maximum(counts, 1.0) + b_ref[0, 0])[None, :]


def kernel(x, batch, table_0, table_1, table_2, table_3, table_4,
           table_5, table_6, table_7, table_8, W, b):
    p = pl.pallas_call(
        _project_body,
        out_shape=jax.ShapeDtypeStruct((176,), jnp.float32),
    )(table_0, table_1, table_2, table_3, table_4, table_5, table_6,
      table_7, table_8, W)

    pool = pl.kernel(
        _pool_body,
        out_type=jax.ShapeDtypeStruct((_NW, 2 * _NSEG), jnp.float32),
        mesh=plsc.VectorSubcoreMesh(core_axis_name="c", subcore_axis_name="s"),
        compiler_params=pltpu.CompilerParams(needs_layout_passes=False),
        scratch_types=[
            pltpu.VMEM((9 * _CHUNK,), jnp.int32),
            pltpu.VMEM((_CHUNK,), jnp.int32),
            pltpu.VMEM((176,), jnp.float32),
            pltpu.VMEM((len(_CONST_ROWS) * 16,), jnp.int32),
            pltpu.VMEM((48,), jnp.float32),
            pltpu.VMEM((64,), jnp.float32),
            pltpu.VMEM((_C012,), jnp.float32),
            pltpu.VMEM((_C345,), jnp.float32),
            pltpu.VMEM((_C678,), jnp.float32),
            pltpu.VMEM((16 * _PITCH,), jnp.float32),
            pltpu.VMEM((16 * _PITCH,), jnp.float32),
            pltpu.VMEM((2 * _NSEG,), jnp.float32),
            pltpu.SemaphoreType.DMA,
            pltpu.SemaphoreType.DMA,
        ],
    )
    partials = pool(x.reshape(-1), batch, p, jnp.asarray(_CONSTS_NP))

    out = pl.pallas_call(
        _finish_body,
        out_shape=jax.ShapeDtypeStruct((1, _G), jnp.float32),
    )(partials, b.reshape(1, 1))
    return out.reshape(_G, 1)


# speedup vs baseline: 2.3731x; 2.3731x over previous
import numpy as np

import jax
import jax.numpy as jnp
from jax import lax
from jax.experimental import pallas as pl
from jax.experimental.pallas import tpu as pltpu
from jax.experimental.pallas import tpu_sc as plsc

_G = 512
_N = 100000
_NW = 32
_CHUNK = 3136
_NVEC = _CHUNK // 16
_LAST_BASE = _N - _CHUNK
_LAST_SKIP = (31 * _CHUNK - _LAST_BASE) // 16
_NSEG = 544
_PITCH = 545

_DIMS = (119, 4, 12, 12, 10, 6, 6, 2, 2)
_OFFS = (0, 119, 123, 135, 147, 157, 163, 169, 171)

_C012 = 119 * 48
_C345 = 12 * 64
_C678 = 32


def _project_body(t0, t1, t2, t3, t4, t5, t6, t7, t8, w, p_ref):
    ts = (t0, t1, t2, t3, t4, t5, t6, t7, t8)
    for t, off, d in zip(ts, _OFFS, _DIMS):
        p_ref[pl.ds(off, d)] = jnp.dot(t[...], w[...])[:, 0]


_CONST_ROWS = []


def _row(vals):
    _CONST_ROWS.append(np.asarray(vals, np.int32))
    return len(_CONST_ROWS) - 1


_R12 = []
for _v in range(3):
    _kk = np.arange(16) + 16 * _v
    _R12.append((_row(_kk // 12 + _OFFS[1]), _row(_kk % 12 + _OFFS[2])))
_R45 = []
for _v in range(4):
    _kk = np.arange(16) + 16 * _v
    _R45.append((_row(np.minimum(_kk // 6, 9) + _OFFS[4]),
                 _row(_kk % 6 + _OFFS[5])))
_R678 = []
for _v in range(2):
    _kk = np.arange(16) + 16 * _v
    _R678.append((_row(np.minimum(_kk // 4, 5) + _OFFS[6]),
                  _row((_kk // 2) % 2 + _OFFS[7]),
                  _row(_kk % 2 + _OFFS[8])))
_CONSTS_NP = np.concatenate(_CONST_ROWS)


def _pool_body(x_hbm, b_hbm, p_hbm, c_hbm, out,
               xv, bv, pv, cv, p12, p45, t012, t345, t678,
               acc_s, acc_c, obuf, sem_big, sem_small):
    wid = lax.axis_index("c") * 16 + lax.axis_index("s")
    is_last = wid == _NW - 1
    base = jnp.where(is_last, _LAST_BASE, wid * _CHUNK)

    cps = [pltpu.async_copy(x_hbm.at[pl.ds(f * _N + base, _CHUNK)],
                            xv.at[pl.ds(f * _CHUNK, _CHUNK)], sem_big)
           for f in range(9)]
    cps.append(pltpu.async_copy(b_hbm.at[pl.ds(base, _CHUNK)], bv, sem_big))
    cp_p = pltpu.async_copy(p_hbm, pv, sem_small)
    cp_c = pltpu.async_copy(c_hbm, cv, sem_small)
    cp_p.wait()
    cp_c.wait()

    iota = lax.iota(jnp.int32, 16)
    zeros = jnp.zeros((16,), jnp.float32)

    def crow(r):
        return cv[pl.ds(16 * r, 16)]

    for v, (ra, rc) in enumerate(_R12):
        p12[pl.ds(16 * v, 16)] = (plsc.load_gather(pv, [crow(ra)])
                                  + plsc.load_gather(pv, [crow(rc)]))
    for v, (ra, rc) in enumerate(_R45):
        p45[pl.ds(16 * v, 16)] = (plsc.load_gather(pv, [crow(ra)])
                                  + plsc.load_gather(pv, [crow(rc)]))
    for v, (ra, rb, rc) in enumerate(_R678):
        t678[pl.ds(16 * v, 16)] = (plsc.load_gather(pv, [crow(ra)])
                                   + plsc.load_gather(pv, [crow(rb)])
                                   + plsc.load_gather(pv, [crow(rc)]))

    def c012_body(a, _):
        s = plsc.load_gather(pv, [jnp.zeros((16,), jnp.int32) + a])
        r = a * 48
        t012[pl.ds(r, 16)] = s + p12[pl.ds(0, 16)]
        t012[pl.ds(r + 16, 16)] = s + p12[pl.ds(16, 16)]
        t012[pl.ds(r + 32, 16)] = s + p12[pl.ds(32, 16)]
        return 0

    lax.fori_loop(0, 119, c012_body, 0)

    def c345_body(a, _):
        s = plsc.load_gather(pv, [jnp.zeros((16,), jnp.int32) + (a + _OFFS[3])])
        r = a * 64
        for v in range(4):
            t345[pl.ds(r + 16 * v, 16)] = s + p45[pl.ds(16 * v, 16)]
        return 0

    lax.fori_loop(0, 12, c345_body, 0)

    def zero_body(k, _):
        acc_s[pl.ds(k * 16, 16)] = zeros
        acc_c[pl.ds(k * 16, 16)] = zeros
        return 0

    lax.fori_loop(0, _PITCH, zero_body, 0)

    for c in cps:
        c.wait()

    ones = jnp.full((16,), 1.0, jnp.float32)
    lane_base = iota * _PITCH

    def step(j):
        off = j * 16
        xf = [xv[pl.ds(f * _CHUNK + off, 16)] for f in range(9)]
        i012 = xf[0] * 48 + xf[1] * 12 + xf[2]
        i345 = xf[3] * 64 + xf[4] * 6 + xf[5]
        i678 = xf[6] * 4 + xf[7] * 2 + xf[8]
        s = plsc.load_gather(t012, [i012])
        s = s + plsc.load_gather(t345, [i345])
        s = s + plsc.load_gather(t678, [i678])
        g = bv[pl.ds(j * 16, 16)] + lane_base
        plsc.addupdate_scatter(acc_s, [g], s)
        plsc.addupdate_scatter(acc_c, [g], ones)

    def body(t, _):
        step(2 * t)
        step(2 * t + 1)
        return 0

    lax.cond(is_last,
             lambda: lax.fori_loop(_LAST_SKIP // 2, _NVEC // 2, body, 0),
             lambda: lax.fori_loop(0, _NVEC // 2, body, 0))

    def red_body(k, _):
        off = k * 16
        ssum = acc_s[pl.ds(off, 16)]
        csum = acc_c[pl.ds(off, 16)]
        for l in range(1, 16):
            ssum = ssum + acc_s[pl.ds(l * _PITCH + off, 16)]
            csum = csum + acc_c[pl.ds(l * _PITCH + off, 16)]
        obuf[pl.ds(off, 16)] = ssum
        obuf[pl.ds(_NSEG + off, 16)] = csum
        return 0

    lax.fori_loop(0, _NSEG // 16, red_body, 0)

    pltpu.sync_copy(obuf, out.at[wid])


def _finish_body(p_ref, b_ref, o_ref):
    tot = jnp.sum(p_ref[...], axis=0)
    sums = tot[:_G]
    counts = tot[_NSEG:_NSEG + _G]
    o_ref[...] = (sums / jnp.maximum(counts, 1.0) + b_ref[0, 0])[None, :]


def kernel(x, batch, table_0, table_1, table_2, table_3, table_4,
           table_5, table_6, table_7, table_8, W, b):
    p = pl.pallas_call(
        _project_body,
        out_shape=jax.ShapeDtypeStruct((176,), jnp.float32),
    )(table_0, table_1, table_2, table_3, table_4, table_5, table_6,
      table_7, table_8, W)

    pool = pl.kernel(
        _pool_body,
        out_type=jax.ShapeDtypeStruct((_NW, 2 * _NSEG), jnp.float32),
        mesh=plsc.VectorSubcoreMesh(core_axis_name="c", subcore_axis_name="s"),
        compiler_params=pltpu.CompilerParams(needs_layout_passes=False),
        scratch_types=[
            pltpu.VMEM((9 * _CHUNK,), jnp.int32),
            pltpu.VMEM((_CHUNK,), jnp.int32),
            pltpu.VMEM((176,), jnp.float32),
            pltpu.VMEM((len(_CONST_ROWS) * 16,), jnp.int32),
            pltpu.VMEM((48,), jnp.float32),
            pltpu.VMEM((64,), jnp.float32),
            pltpu.VMEM((_C012,), jnp.float32),
            pltpu.VMEM((_C345,), jnp.float32),
            pltpu.VMEM((_C678,), jnp.float32),
            pltpu.VMEM((16 * _PITCH,), jnp.float32),
            pltpu.VMEM((16 * _PITCH,), jnp.float32),
            pltpu.VMEM((2 * _NSEG,), jnp.float32),
            pltpu.SemaphoreType.DMA,
            pltpu.SemaphoreType.DMA,
        ],
    )
    partials = pool(x.T.reshape(-1), batch, p, jnp.asarray(_CONSTS_NP))

    out = pl.pallas_call(
        _finish_body,
        out_shape=jax.ShapeDtypeStruct((1, _G), jnp.float32),
    )(partials, b.reshape(1, 1))
    return out.reshape(_G, 1)
